# baseline (speedup 1.0000x reference)
import functools

import jax
import jax.numpy as jnp
from jax.experimental import pallas as pl
from jax.experimental.pallas import tpu as pltpu

_SLOPE = (1.0 / 8.0 + 1.0 / 3.0) / 2.0
_B = 16
_BLK = 4096

_PI_HI = 3.1415927410125732
_PI_LO = -8.742277657347586e-08
_INV_PI = 0.3183098861837907


def _rr(x):
    return jnp.where(x >= 0, x, x * _SLOPE)


def _sin_poly(a):
    k = jnp.round(a * _INV_PI)
    r = (a - k * _PI_HI) - k * _PI_LO
    r2 = r * r
    s = r * (1.0 + r2 * (-1.66666667e-01 + r2 * (8.33333376e-03
             + r2 * (-1.98412698e-04 + r2 * 2.75573192e-06))))
    odd = jax.lax.bitwise_and(k.astype(jnp.int32), 1)
    return jnp.where(odd == 1, -s, s)


def _dot(a, b):
    return jax.lax.dot_general(a, b, (((1,), (0,)), ((), ())),
                               preferred_element_type=jnp.float32)


def _dott(a, b):
    return jax.lax.dot_general(a, b, (((0,), (0,)), ((), ())),
                               preferred_element_type=jnp.float32)


def _aggr_kernel(nb, t2v_d,
                 auxt_ref, xall_ref,
                 s24t_ref, iota48_ref, r48_ref,
                 w1t_fr_ref, w1frx_ref, b1fr_ref, w2fr_ref, b2fr_ref,
                 w1t_m_ref, w1mx_ref, b1m_ref, w2m_ref, b2m_ref,
                 wcf_ref, wcr_ref, wcm_ref, bc_ref,
                 out_ref,
                 acc):
    i = pl.program_id(0)

    @pl.when(i == 0)
    def _init():
        acc[:, :] = jnp.zeros_like(acc)

    auxt = auxt_ref[:, :]

    a_t = _dot(s24t_ref[:, :], auxt[0:4, :])
    row = jax.lax.broadcasted_iota(jnp.int32, a_t.shape, 0)
    t2v_t = jnp.where(row % t2v_d == 0, a_t, _sin_poly(a_t))

    xall = xall_ref[:, :]
    h_fr = _rr(_dott(t2v_t, w1t_fr_ref[:, :])
               + _dot(xall, w1frx_ref[:, :]) + b1fr_ref[:, :])
    h_fr = _rr(_dot(h_fr, w2fr_ref[:, :]) + b2fr_ref[:, :])

    h_m = _rr(_dott(t2v_t, w1t_m_ref[:, :])
              + _dot(xall, w1mx_ref[:, :]) + b1m_ref[:, :])
    h_m = _rr(_dot(h_m, w2m_ref[:, :]) + b2m_ref[:, :])

    seg_rep = _dott(auxt[4:7, :], r48_ref[:, :])
    oh = (seg_rep == iota48_ref[:, :]).astype(jnp.float32)

    ones_col = jnp.ones((seg_rep.shape[0], 1), jnp.float32)
    acc[:, 0:h_fr.shape[1]] += _dott(oh, h_fr)
    acc[:, 128:128 + h_m.shape[1]] += _dott(oh, h_m)
    acc[:, 192:193] += _dott(oh, ones_col)

    @pl.when(i == nb - 1)
    def _fin():
        s = acc[:, :]
        o = h_m.shape[1]
        mf = s[0:_B, 0:o] / jnp.maximum(s[0:_B, 192:193], 1.0)
        mr = s[_B:2 * _B, o:2 * o] / jnp.maximum(s[_B:2 * _B, 192:193], 1.0)
        mm = s[2 * _B:3 * _B, 128:128 + o] / jnp.maximum(
            s[2 * _B:3 * _B, 192:193], 1.0)
        y = (_dot(mf, wcf_ref[:, :]) + _dot(mr, wcr_ref[:, :])
             + _dot(mm, wcm_ref[:, :]) + bc_ref[:, :])
        out_ref[:, :] = _rr(y)


def kernel(t_f, f_feats, seg_f, t_r, r_feats, seg_r, t_m, m_feats, seg_m,
           W_t2v, b_t2v, Wf1, bf1, Wf2, bf2, Wr1, br1, Wr2, br2,
           Wm1, bm1, Wm2, bm2, W_combo, b_combo):
    n = t_f.shape[0]
    blk = _BLK
    while n % blk:
        blk //= 2
    nb = n // blk
    t2v_d = W_t2v.shape[0]
    out_dim = W_combo.shape[0]
    hid = Wf1.shape[0]
    o3 = Wf2.shape[0]
    nf = f_feats.shape[1]

    rowv = lambda a: a.reshape(1, n)
    auxt = jnp.concatenate(
        [rowv(t_f), rowv(t_r), rowv(t_m),
         jnp.ones((1, n), jnp.float32),
         rowv(seg_f.astype(jnp.float32)),
         rowv(seg_r.astype(jnp.float32)),
         rowv(seg_m.astype(jnp.float32)),
         jnp.zeros((1, n), jnp.float32)], axis=0)

    w_row = W_t2v.reshape(t2v_d)
    z8 = jnp.zeros((t2v_d,), jnp.float32)
    s24t = jnp.stack([
        jnp.concatenate([w_row, z8, z8]),
        jnp.concatenate([z8, w_row, z8]),
        jnp.concatenate([z8, z8, w_row]),
        jnp.concatenate([b_t2v, b_t2v, b_t2v]),
    ], axis=1)

    zb = jnp.zeros((_B,), jnp.float32)
    ob = jnp.ones((_B,), jnp.float32)
    r48 = jnp.stack([
        jnp.concatenate([ob, zb, zb]),
        jnp.concatenate([zb, ob, zb]),
        jnp.concatenate([zb, zb, ob]),
    ], axis=0)
    iota48 = jnp.tile(jnp.arange(_B, dtype=jnp.float32), 3).reshape(1, 3 * _B)

    zt = jnp.zeros((t2v_d, hid), jnp.float32)
    w1t_fr = jnp.concatenate([
        jnp.concatenate([Wf1[:, :t2v_d].T, zt], axis=1),
        jnp.concatenate([zt, Wr1[:, :t2v_d].T], axis=1),
        jnp.zeros((t2v_d, 2 * hid), jnp.float32)], axis=0)
    zx = jnp.zeros((nf, hid), jnp.float32)
    w1frx = jnp.concatenate([
        jnp.concatenate([Wf1[:, t2v_d:].T, zx], axis=1),
        jnp.concatenate([zx, Wr1[:, t2v_d:].T], axis=1),
        jnp.zeros((nf, 2 * hid), jnp.float32)],
        axis=0).astype(jnp.bfloat16)
    b1fr = jnp.concatenate([bf1, br1]).reshape(1, -1)
    zh = jnp.zeros((hid, o3), jnp.float32)
    w2fr = jnp.concatenate([
        jnp.concatenate([Wf2.T, zh], axis=1),
        jnp.concatenate([zh, Wr2.T], axis=1)], axis=0)
    b2fr = jnp.concatenate([bf2, br2]).reshape(1, -1)

    w1t_m = jnp.concatenate([
        jnp.zeros((2 * t2v_d, hid), jnp.float32),
        Wm1[:, :t2v_d].T], axis=0)
    w1mx = jnp.concatenate([
        jnp.zeros((2 * nf, hid), jnp.float32),
        Wm1[:, t2v_d:].T], axis=0).astype(jnp.bfloat16)
    b1m = bm1.reshape(1, -1)
    w2m = Wm2.T
    b2m = bm2.reshape(1, -1)

    wcf = W_combo[:, :o3].T
    wcr = W_combo[:, o3:2 * o3].T
    wcm = W_combo[:, 2 * o3:].T
    bc = b_combo.reshape(1, -1)

    tokc = lambda i: (0, i)
    tok = lambda i: (i, 0)
    fix = lambda i: (0, 0)

    def xspec(w):
        return pl.BlockSpec((blk, w), tok)

    def wspec(a):
        return pl.BlockSpec(a.shape, fix)

    xall = jnp.concatenate(
        [f_feats.astype(jnp.bfloat16), r_feats.astype(jnp.bfloat16),
         m_feats.astype(jnp.bfloat16)], axis=1)

    in_specs = [
        pl.BlockSpec((8, blk), tokc),
        xspec(3 * nf),
        wspec(s24t), wspec(iota48), wspec(r48),
        wspec(w1t_fr), wspec(w1frx), wspec(b1fr),
        wspec(w2fr), wspec(b2fr),
        wspec(w1t_m), wspec(w1mx), wspec(b1m), wspec(w2m), wspec(b2m),
        wspec(wcf), wspec(wcr), wspec(wcm), wspec(bc),
    ]

    out = pl.pallas_call(
        functools.partial(_aggr_kernel, nb, t2v_d),
        grid=(nb,),
        in_specs=in_specs,
        out_specs=pl.BlockSpec((_B, out_dim), fix),
        out_shape=jax.ShapeDtypeStruct((_B, out_dim), jnp.float32),
        scratch_shapes=[
            pltpu.VMEM((3 * _B, 193), jnp.float32),
        ],
        compiler_params=pltpu.CompilerParams(
            dimension_semantics=("arbitrary",)),
    )(auxt, xall,
      s24t, iota48, r48,
      w1t_fr, w1frx, b1fr, w2fr, b2fr,
      w1t_m, w1mx, b1m, w2m, b2m,
      wcf, wcr, wcm, bc)
    return out


# speedup vs baseline: 1.1753x; 1.1753x over previous
import functools

import jax
import jax.numpy as jnp
from jax.experimental import pallas as pl
from jax.experimental.pallas import tpu as pltpu

_SLOPE = (1.0 / 8.0 + 1.0 / 3.0) / 2.0
_B = 16
_BLK = 4096

_PI_HI = 3.1415927410125732
_PI_LO = -8.742277657347586e-08
_INV_PI = 0.3183098861837907


def _rr(x):
    return jnp.where(x >= 0, x, x * _SLOPE)


def _sin_poly(a):
    k = jnp.round(a * _INV_PI)
    r = (a - k * _PI_HI) - k * _PI_LO
    r2 = r * r
    s = r * (1.0 + r2 * (-1.66666667e-01 + r2 * (8.33333376e-03
             + r2 * (-1.98412698e-04 + r2 * 2.75573192e-06))))
    odd = jax.lax.bitwise_and(k.astype(jnp.int32), 1)
    return jnp.where(odd == 1, -s, s)


def _dot(a, b):
    return jax.lax.dot_general(a, b, (((1,), (0,)), ((), ())),
                               preferred_element_type=jnp.float32)


def _dott(a, b):
    return jax.lax.dot_general(a, b, (((0,), (0,)), ((), ())),
                               preferred_element_type=jnp.float32)


def _aggr_kernel(nb, t2v_d,
                 auxt_ref, xf_ref, xr_ref, xm_ref,
                 s24t_ref, iota48_ref, r48_ref,
                 w1t_fr_ref, w1f_ref, w1r_ref, b1fr_ref, w2fr_ref, b2fr_ref,
                 w1t_m_ref, w1m_ref, b1m_ref, w2m_ref, b2m_ref,
                 wcf_ref, wcr_ref, wcm_ref, bc_ref,
                 out_ref,
                 acc):
    i = pl.program_id(0)

    @pl.when(i == 0)
    def _init():
        acc[:, :] = jnp.zeros_like(acc)

    auxt = auxt_ref[:, :]

    a_t = _dot(s24t_ref[:, :], auxt[0:4, :])
    row = jax.lax.broadcasted_iota(jnp.int32, a_t.shape, 0)
    t2v_t = jnp.where(row % t2v_d == 0, a_t, _sin_poly(a_t))

    h_fr = _rr(_dott(t2v_t, w1t_fr_ref[:, :])
               + _dot(xf_ref[:, :], w1f_ref[:, :])
               + _dot(xr_ref[:, :], w1r_ref[:, :]) + b1fr_ref[:, :])
    h_fr = _rr(_dot(h_fr, w2fr_ref[:, :]) + b2fr_ref[:, :])

    h_m = _rr(_dott(t2v_t, w1t_m_ref[:, :])
              + _dot(xm_ref[:, :], w1m_ref[:, :]) + b1m_ref[:, :])
    h_m = _rr(_dot(h_m, w2m_ref[:, :]) + b2m_ref[:, :])

    seg_rep = _dott(auxt[4:7, :], r48_ref[:, :])
    oh = (seg_rep == iota48_ref[:, :]).astype(jnp.float32)

    ones_col = jnp.ones((seg_rep.shape[0], 1), jnp.float32)
    acc[:, 0:h_fr.shape[1]] += _dott(oh, h_fr)
    acc[:, 128:128 + h_m.shape[1]] += _dott(oh, h_m)
    acc[:, 192:193] += _dott(oh, ones_col)

    @pl.when(i == nb - 1)
    def _fin():
        s = acc[:, :]
        o = h_m.shape[1]
        mf = s[0:_B, 0:o] / jnp.maximum(s[0:_B, 192:193], 1.0)
        mr = s[_B:2 * _B, o:2 * o] / jnp.maximum(s[_B:2 * _B, 192:193], 1.0)
        mm = s[2 * _B:3 * _B, 128:128 + o] / jnp.maximum(
            s[2 * _B:3 * _B, 192:193], 1.0)
        y = (_dot(mf, wcf_ref[:, :]) + _dot(mr, wcr_ref[:, :])
             + _dot(mm, wcm_ref[:, :]) + bc_ref[:, :])
        out_ref[:, :] = _rr(y)


def kernel(t_f, f_feats, seg_f, t_r, r_feats, seg_r, t_m, m_feats, seg_m,
           W_t2v, b_t2v, Wf1, bf1, Wf2, bf2, Wr1, br1, Wr2, br2,
           Wm1, bm1, Wm2, bm2, W_combo, b_combo):
    n = t_f.shape[0]
    blk = _BLK
    while n % blk:
        blk //= 2
    nb = n // blk
    t2v_d = W_t2v.shape[0]
    out_dim = W_combo.shape[0]
    hid = Wf1.shape[0]
    o3 = Wf2.shape[0]
    nf = f_feats.shape[1]

    rowv = lambda a: a.reshape(1, n)
    auxt = jnp.concatenate(
        [rowv(t_f), rowv(t_r), rowv(t_m),
         jnp.ones((1, n), jnp.float32),
         rowv(seg_f.astype(jnp.float32)),
         rowv(seg_r.astype(jnp.float32)),
         rowv(seg_m.astype(jnp.float32)),
         jnp.zeros((1, n), jnp.float32)], axis=0)

    w_row = W_t2v.reshape(t2v_d)
    z8 = jnp.zeros((t2v_d,), jnp.float32)
    s24t = jnp.stack([
        jnp.concatenate([w_row, z8, z8]),
        jnp.concatenate([z8, w_row, z8]),
        jnp.concatenate([z8, z8, w_row]),
        jnp.concatenate([b_t2v, b_t2v, b_t2v]),
    ], axis=1)

    zb = jnp.zeros((_B,), jnp.float32)
    ob = jnp.ones((_B,), jnp.float32)
    r48 = jnp.stack([
        jnp.concatenate([ob, zb, zb]),
        jnp.concatenate([zb, ob, zb]),
        jnp.concatenate([zb, zb, ob]),
    ], axis=0)
    iota48 = jnp.tile(jnp.arange(_B, dtype=jnp.float32), 3).reshape(1, 3 * _B)

    zt = jnp.zeros((t2v_d, hid), jnp.float32)
    w1t_fr = jnp.concatenate([
        jnp.concatenate([Wf1[:, :t2v_d].T, zt], axis=1),
        jnp.concatenate([zt, Wr1[:, :t2v_d].T], axis=1),
        jnp.zeros((t2v_d, 2 * hid), jnp.float32)], axis=0)
    zx = jnp.zeros((nf, hid), jnp.float32)
    w1f = jnp.concatenate([Wf1[:, t2v_d:].T, zx],
                          axis=1).astype(jnp.bfloat16)
    w1r = jnp.concatenate([zx, Wr1[:, t2v_d:].T],
                          axis=1).astype(jnp.bfloat16)
    b1fr = jnp.concatenate([bf1, br1]).reshape(1, -1)
    zh = jnp.zeros((hid, o3), jnp.float32)
    w2fr = jnp.concatenate([
        jnp.concatenate([Wf2.T, zh], axis=1),
        jnp.concatenate([zh, Wr2.T], axis=1)], axis=0)
    b2fr = jnp.concatenate([bf2, br2]).reshape(1, -1)

    w1t_m = jnp.concatenate([
        jnp.zeros((2 * t2v_d, hid), jnp.float32),
        Wm1[:, :t2v_d].T], axis=0)
    w1m = Wm1[:, t2v_d:].T.astype(jnp.bfloat16)
    b1m = bm1.reshape(1, -1)
    w2m = Wm2.T
    b2m = bm2.reshape(1, -1)

    wcf = W_combo[:, :o3].T
    wcr = W_combo[:, o3:2 * o3].T
    wcm = W_combo[:, 2 * o3:].T
    bc = b_combo.reshape(1, -1)

    tokc = lambda i: (0, i)
    tok = lambda i: (i, 0)
    fix = lambda i: (0, 0)

    def xspec(w):
        return pl.BlockSpec((blk, w), tok)

    def wspec(a):
        return pl.BlockSpec(a.shape, fix)

    in_specs = [
        pl.BlockSpec((8, blk), tokc),
        xspec(nf), xspec(r_feats.shape[1]), xspec(m_feats.shape[1]),
        wspec(s24t), wspec(iota48), wspec(r48),
        wspec(w1t_fr), wspec(w1f), wspec(w1r), wspec(b1fr),
        wspec(w2fr), wspec(b2fr),
        wspec(w1t_m), wspec(w1m), wspec(b1m), wspec(w2m), wspec(b2m),
        wspec(wcf), wspec(wcr), wspec(wcm), wspec(bc),
    ]

    out = pl.pallas_call(
        functools.partial(_aggr_kernel, nb, t2v_d),
        grid=(nb,),
        in_specs=in_specs,
        out_specs=pl.BlockSpec((_B, out_dim), fix),
        out_shape=jax.ShapeDtypeStruct((_B, out_dim), jnp.float32),
        scratch_shapes=[
            pltpu.VMEM((3 * _B, 193), jnp.float32),
        ],
        compiler_params=pltpu.CompilerParams(
            dimension_semantics=("arbitrary",)),
    )(auxt, f_feats.astype(jnp.bfloat16), r_feats.astype(jnp.bfloat16),
      m_feats.astype(jnp.bfloat16),
      s24t, iota48, r48,
      w1t_fr, w1f, w1r, b1fr, w2fr, b2fr,
      w1t_m, w1m, b1m, w2m, b2m,
      wcf, wcr, wcm, bc)
    return out
